# Initial kernel scaffold; baseline (speedup 1.0000x reference)
import jax
import jax.numpy as jnp
from jax.experimental import pallas as pl


def kernel(table, list_data):
    raise NotImplementedError("write your pallas kernel here")


# speedup vs baseline: 1.5523x; 1.5523x over previous
import functools

import jax
import jax.numpy as jnp
from jax import lax
from jax.experimental import pallas as pl
from jax.experimental.pallas import tpu as pltpu
from jax.experimental.pallas import tpu_sc as plsc

_EMBED_DIM = 32
_IDX_PER_STREAM = 128
_CHUNK = 1024


@functools.lru_cache(maxsize=None)
def _build_gather(n_rows_total, dim):
    info = plsc.get_sparse_core_info()
    nc, ns = info.num_cores, info.num_subcores
    nw = nc * ns
    assert n_rows_total % (nw * _CHUNK) == 0
    b_per_w = n_rows_total // nw
    n_chunks = b_per_w // _CHUNK
    mesh = plsc.VectorSubcoreMesh(core_axis_name="c", subcore_axis_name="s")

    @functools.partial(
        pl.kernel,
        mesh=mesh,
        out_type=jax.ShapeDtypeStruct((n_rows_total, dim), jnp.float32),
        scratch_types=[
            pltpu.VMEM((_CHUNK,), jnp.int32),
            pltpu.VMEM((_CHUNK, dim), jnp.float32),
            pltpu.SemaphoreType.DMA,
        ],
        compiler_params=pltpu.CompilerParams(use_tc_tiling_on_sc=False),
    )
    def gather_kernel(table_hbm, idx_hbm, out_hbm, idx_v, rows_v, sem):
        wid = lax.axis_index("s") * nc + lax.axis_index("c")
        base = wid * b_per_w

        def body(g, carry):
            off = base + g * _CHUNK
            pltpu.sync_copy(idx_hbm.at[pl.ds(off, _CHUNK)], idx_v)
            copies = []
            for j in range(_CHUNK // _IDX_PER_STREAM):
                s = j * _IDX_PER_STREAM
                copies.append(
                    pltpu.async_copy(
                        table_hbm.at[idx_v.at[pl.ds(s, _IDX_PER_STREAM)]],
                        rows_v.at[pl.ds(s, _IDX_PER_STREAM), :],
                        sem,
                    )
                )
            for c in copies:
                c.wait()
            pltpu.sync_copy(rows_v, out_hbm.at[pl.ds(off, _CHUNK)])
            return carry

        lax.fori_loop(0, n_chunks, body, 0)

    return gather_kernel


def kernel(table, list_data):
    idx = list_data.reshape(-1).astype(jnp.int32)
    out = _build_gather(idx.shape[0], table.shape[1])(table, idx)
    return out.reshape(*list_data.shape, table.shape[1])


# speedup vs baseline: 1.5800x; 1.0178x over previous
import functools

import jax
import jax.numpy as jnp
from jax import lax
from jax.experimental import pallas as pl
from jax.experimental.pallas import tpu as pltpu
from jax.experimental.pallas import tpu_sc as plsc

_EMBED_DIM = 32
_N_CHUNKS = 8


@functools.lru_cache(maxsize=None)
def _build_gather(n_rows_total, dim):
    info = plsc.get_sparse_core_info()
    nc, ns = info.num_cores, info.num_subcores
    nw = nc * ns
    assert n_rows_total % (nw * _N_CHUNKS) == 0
    b_per_w = n_rows_total // nw
    chunk = b_per_w // _N_CHUNKS
    mesh = plsc.VectorSubcoreMesh(core_axis_name="c", subcore_axis_name="s")

    @functools.partial(
        pl.kernel,
        mesh=mesh,
        out_type=jax.ShapeDtypeStruct((n_rows_total, dim), jnp.float32),
        scratch_types=[
            pltpu.VMEM((b_per_w,), jnp.int32),
            pltpu.VMEM((chunk, dim), jnp.float32),
            pltpu.VMEM((chunk, dim), jnp.float32),
            pltpu.SemaphoreType.DMA,
            pltpu.SemaphoreType.DMA,
            pltpu.SemaphoreType.DMA,
            pltpu.SemaphoreType.DMA,
        ],
        compiler_params=pltpu.CompilerParams(use_tc_tiling_on_sc=False),
    )
    def gather_kernel(table_hbm, idx_hbm, out_hbm, idx_v, buf0, buf1,
                      sg0, sg1, ss0, ss1):
        wid = lax.axis_index("s") * nc + lax.axis_index("c")
        base = wid * b_per_w
        pltpu.sync_copy(idx_hbm.at[pl.ds(base, b_per_w)], idx_v)

        bufs = (buf0, buf1)
        gsems = (sg0, sg1)
        ssems = (ss0, ss1)
        gathers = [None] * _N_CHUNKS
        stores = [None] * _N_CHUNKS

        def fire_gather(c):
            b = c & 1
            return pltpu.async_copy(
                table_hbm.at[idx_v.at[pl.ds(c * chunk, chunk)]],
                bufs[b], gsems[b])

        def fire_store(c):
            b = c & 1
            return pltpu.async_copy(
                bufs[b], out_hbm.at[pl.ds(base + c * chunk, chunk)], ssems[b])

        for c in range(_N_CHUNKS):
            if c >= 2:
                stores[c - 2].wait()
            gathers[c] = fire_gather(c)
            if c >= 1:
                gathers[c - 1].wait()
                stores[c - 1] = fire_store(c - 1)
        stores[_N_CHUNKS - 2].wait()
        gathers[_N_CHUNKS - 1].wait()
        stores[_N_CHUNKS - 1] = fire_store(_N_CHUNKS - 1)
        stores[_N_CHUNKS - 1].wait()

    return gather_kernel


def kernel(table, list_data):
    idx = list_data.reshape(-1).astype(jnp.int32)
    out = _build_gather(idx.shape[0], table.shape[1])(table, idx)
    return out.reshape(*list_data.shape, table.shape[1])


# speedup vs baseline: 1.5806x; 1.0004x over previous
import functools

import jax
import jax.numpy as jnp
from jax import lax
from jax.experimental import pallas as pl
from jax.experimental.pallas import tpu as pltpu
from jax.experimental.pallas import tpu_sc as plsc

_EMBED_DIM = 32
_N_CHUNKS = 16
_N_BUFS = 4
_DEPTH = 3


@functools.lru_cache(maxsize=None)
def _build_gather(n_rows_total, dim):
    info = plsc.get_sparse_core_info()
    nc, ns = info.num_cores, info.num_subcores
    nw = nc * ns
    assert n_rows_total % (nw * _N_CHUNKS) == 0
    b_per_w = n_rows_total // nw
    chunk = b_per_w // _N_CHUNKS
    mesh = plsc.VectorSubcoreMesh(core_axis_name="c", subcore_axis_name="s")

    scratch = [pltpu.VMEM((b_per_w,), jnp.int32)]
    scratch += [pltpu.VMEM((chunk, dim), jnp.float32)] * _N_BUFS
    scratch += [pltpu.SemaphoreType.DMA] * (2 * _N_BUFS)

    @functools.partial(
        pl.kernel,
        mesh=mesh,
        out_type=jax.ShapeDtypeStruct((n_rows_total, dim), jnp.float32),
        scratch_types=scratch,
        compiler_params=pltpu.CompilerParams(use_tc_tiling_on_sc=False),
    )
    def gather_kernel(table_hbm, idx_hbm, out_hbm, idx_v, *bufs_sems):
        bufs = bufs_sems[:_N_BUFS]
        gsems = bufs_sems[_N_BUFS:2 * _N_BUFS]
        ssems = bufs_sems[2 * _N_BUFS:]
        wid = lax.axis_index("s") * nc + lax.axis_index("c")
        base = wid * b_per_w
        pltpu.sync_copy(idx_hbm.at[pl.ds(base, b_per_w)], idx_v)

        gathers = [None] * _N_CHUNKS
        stores = [None] * _N_CHUNKS

        def fire_gather(c):
            b = c % _N_BUFS
            return pltpu.async_copy(
                table_hbm.at[idx_v.at[pl.ds(c * chunk, chunk)]],
                bufs[b], gsems[b])

        def fire_store(c):
            b = c % _N_BUFS
            return pltpu.async_copy(
                bufs[b], out_hbm.at[pl.ds(base + c * chunk, chunk)], ssems[b])

        for c in range(_N_CHUNKS):
            if c >= _N_BUFS:
                stores[c - _N_BUFS].wait()
            gathers[c] = fire_gather(c)
            if c >= _DEPTH - 1:
                s = c - (_DEPTH - 1)
                gathers[s].wait()
                stores[s] = fire_store(s)
        for s in range(_N_CHUNKS - (_DEPTH - 1), _N_CHUNKS):
            gathers[s].wait()
            stores[s] = fire_store(s)
        for s in range(_N_CHUNKS - _N_BUFS, _N_CHUNKS):
            stores[s].wait()

    return gather_kernel


def kernel(table, list_data):
    idx = list_data.reshape(-1).astype(jnp.int32)
    out = _build_gather(idx.shape[0], table.shape[1])(table, idx)
    return out.reshape(*list_data.shape, table.shape[1])
